# baseline (speedup 1.0000x reference)
import functools

import jax
import jax.numpy as jnp
from jax import lax
from jax.experimental import pallas as pl
from jax.experimental.pallas import tpu as pltpu
from jax.experimental.pallas import tpu_sc as plsc

_S = 64.0
_M3 = 0.35

_NC = 2
_NS = 16
_LANES = 16


def _sc_gather_finals(logits, labels):
    B, _ = logits.shape
    nw = _NC * _NS
    per_w = B // nw

    mesh = plsc.VectorSubcoreMesh(
        core_axis_name="c", subcore_axis_name="s",
        num_cores=_NC, num_subcores=_NS,
    )

    @functools.partial(
        pl.kernel,
        out_type=jax.ShapeDtypeStruct((B,), jnp.float32),
        mesh=mesh,
        compiler_params=pltpu.CompilerParams(needs_layout_passes=False),
        scratch_types=[
            pltpu.VMEM((per_w,), jnp.int32),
            pltpu.VMEM((per_w, _LANES), jnp.float32),
            pltpu.VMEM((_LANES,), jnp.float32),
            pltpu.SemaphoreType.DMA,
        ],
    )
    def body(x_hbm, labels_hbm, out_hbm, lab_v, win_v, fin_v, sem):
        wid = lax.axis_index("s") * _NC + lax.axis_index("c")
        base = wid * per_w
        pltpu.sync_copy(labels_hbm.at[pl.ds(base, per_w)], lab_v)
        iota = lax.iota(jnp.int32, _LANES)

        descs = []
        for j in range(per_w):
            chunk = lab_v[pl.ds((j // _LANES) * _LANES, _LANES)]
            lab_j = jnp.sum(jnp.where(iota == (j % _LANES), chunk, 0))
            w0 = (lab_j // _LANES) * _LANES
            descs.append(pltpu.async_copy(
                x_hbm.at[base + j, pl.ds(w0, _LANES)], win_v.at[j], sem))
        for d in descs:
            d.wait()

        for k in range(per_w // _LANES):
            sl = pl.ds(k * _LANES, _LANES)
            lanes = lab_v[sl] - (lab_v[sl] // _LANES) * _LANES
            t = plsc.load_gather(win_v, [iota + k * _LANES, lanes])
            fin_v[...] = (t - _M3) * _S
            pltpu.sync_copy(fin_v, out_hbm.at[pl.ds(base + k * _LANES, _LANES)])

    return body(logits, labels)


def _tc_scale_scatter(logits, labels2d, finals2d, block_r):
    n_rows, n_cols = logits.shape
    grid = (n_rows // block_r,)

    def body(x_ref, lab_ref, fin_ref, o_ref):
        col = lax.broadcasted_iota(jnp.int32, (block_r, n_cols), 1)
        mask = col == lab_ref[...]
        o_ref[...] = jnp.where(mask, fin_ref[...], x_ref[...] * _S)

    return pl.pallas_call(
        body,
        grid=grid,
        in_specs=[
            pl.BlockSpec((block_r, n_cols), lambda i: (i, 0)),
            pl.BlockSpec((block_r, 1), lambda i: (i, 0)),
            pl.BlockSpec((block_r, 1), lambda i: (i, 0)),
        ],
        out_specs=pl.BlockSpec((block_r, n_cols), lambda i: (i, 0)),
        out_shape=jax.ShapeDtypeStruct((n_rows, n_cols), jnp.float32),
        compiler_params=pltpu.CompilerParams(
            dimension_semantics=("arbitrary",),
        ),
    )(logits, labels2d, finals2d)


def kernel(logits, labels):
    B, V = logits.shape
    labels = labels.astype(jnp.int32)
    finals = _sc_gather_finals(logits, labels)
    return _tc_scale_scatter(
        logits, labels.reshape(B, 1), finals.reshape(B, 1), 16
    )
